# baseline (speedup 1.0000x reference)
import functools

import jax
import jax.numpy as jnp
from jax import lax
from jax.experimental import pallas as pl
from jax.experimental.pallas import tpu as pltpu
from jax.experimental.pallas import tpu_sc as plsc

EMB = 64
HID = 256
OUT = 48
B = 16384
WIN = 5
WINP = 6

NW = 32
ROWS = B * WINP
ROWS_PER_W = ROWS // NW
CHUNK = 128
NCHUNK = ROWS_PER_W // CHUNK


@functools.cache
def _build_sc_gather():
    mesh = plsc.VectorSubcoreMesh(core_axis_name="c", subcore_axis_name="s")

    @functools.partial(
        pl.kernel,
        out_type=jax.ShapeDtypeStruct((ROWS, EMB), jnp.float32),
        mesh=mesh,
        scratch_types=[
            pltpu.VMEM((NCHUNK, CHUNK), jnp.int32),
            pltpu.VMEM((2, CHUNK, EMB), jnp.float32),
            pltpu.SemaphoreType.DMA,
            pltpu.SemaphoreType.DMA,
            pltpu.SemaphoreType.DMA,
        ],
        compiler_params=pltpu.CompilerParams(use_tc_tiling_on_sc=False),
    )
    def _sc_gather(x_hbm, table_hbm, out_hbm, idx_v, rows_v, gsem, osem0, osem1):
        wid = lax.axis_index("s") * 2 + lax.axis_index("c")
        pltpu.sync_copy(x_hbm.at[wid], idx_v)
        osems = (osem0, osem1)
        base = wid * ROWS_PER_W
        for j in range(NCHUNK):
            slot = j % 2
            buf = rows_v.at[slot]
            gather = pltpu.async_copy(table_hbm.at[idx_v.at[j]], buf, gsem)
            if j >= 2:
                pltpu.make_async_copy(
                    rows_v.at[slot],
                    out_hbm.at[pl.ds(base + (j - 2) * CHUNK, CHUNK)],
                    osems[slot],
                ).wait()
            gather.wait()
            pltpu.async_copy(
                buf, out_hbm.at[pl.ds(base + j * CHUNK, CHUNK)], osems[slot]
            )
        for j in (NCHUNK - 2, NCHUNK - 1):
            slot = j % 2
            pltpu.make_async_copy(
                rows_v.at[slot], out_hbm.at[pl.ds(base + j * CHUNK, CHUNK)],
                osems[slot],
            ).wait()

    return _sc_gather


def _mlp_body(flat_ref, w1_ref, b1_ref, w2_ref, b2_ref, out_ref):
    nrow = flat_ref.shape[0] * 8
    acc = b1_ref[...]
    for t in range(3):
        xt = flat_ref[:, t, :, :].reshape(nrow, 128)
        acc = acc + jnp.dot(
            xt, w1_ref[t], preferred_element_type=jnp.float32
        )
    h = jnp.tanh(acc)
    out_ref[...] = (
        jnp.dot(h, w2_ref[...], preferred_element_type=jnp.float32) + b2_ref[...]
    )


BLKR = 256


@jax.jit
def kernel(x, table, W1, b1, W2, b2):
    x6 = jnp.concatenate([x, jnp.zeros((B, 1), jnp.int32)], axis=1)
    idx_perm = x6.reshape(B // 8, 8, 3, 2).transpose(0, 2, 1, 3)
    idx_perm = idx_perm.reshape(NW, NCHUNK, CHUNK)
    flat = _build_sc_gather()(idx_perm, table)
    flat4d = flat.reshape(B // 8, 3, 8, 128)
    w1p = jnp.concatenate([W1, jnp.zeros((64, HID), jnp.float32)], axis=0)
    w1p = w1p.reshape(3, 128, HID)
    out = pl.pallas_call(
        _mlp_body,
        grid=(B // (8 * BLKR),),
        in_specs=[
            pl.BlockSpec((BLKR, 3, 8, 128), lambda i: (i, 0, 0, 0)),
            pl.BlockSpec((3, 128, HID), lambda i: (0, 0, 0)),
            pl.BlockSpec((1, HID), lambda i: (0, 0)),
            pl.BlockSpec((HID, OUT), lambda i: (0, 0)),
            pl.BlockSpec((1, OUT), lambda i: (0, 0)),
        ],
        out_specs=pl.BlockSpec((8 * BLKR, OUT), lambda i: (i, 0)),
        out_shape=jax.ShapeDtypeStruct((B, OUT), jnp.float32),
    )(flat4d, w1p, b1.reshape(1, HID), W2, b2.reshape(1, OUT))
    return out


# speedup vs baseline: 3.6449x; 3.6449x over previous
import functools

import numpy as np
import jax
import jax.numpy as jnp
from jax import lax
from jax.experimental import pallas as pl
from jax.experimental.pallas import tpu as pltpu
from jax.experimental.pallas import tpu_sc as plsc

EMB = 64
HID = 256
OUT = 48
B = 16384
WIN = 5

NW = 32
ROWS = B * 6
ROWS_PER_W = ROWS // NW
CHUNK = 128
NCHUNK = ROWS_PER_W // CHUNK
B_PER_W = B // NW


def _pattern() -> np.ndarray:
    k = np.arange(ROWS_PER_W)
    rb = k // 48
    rem = k % 48
    t = rem // 16
    r = (rem % 16) // 2
    h = rem % 2
    w = np.minimum(2 * t + h, 4)
    return ((8 * rb + r) * WIN + w).astype(np.int32)


_PATTERN = _pattern()


@functools.cache
def _build_sc_gather():
    mesh = plsc.VectorSubcoreMesh(core_axis_name="c", subcore_axis_name="s")

    @functools.partial(
        pl.kernel,
        out_type=jax.ShapeDtypeStruct((ROWS, EMB), jnp.float32),
        mesh=mesh,
        scratch_types=[
            pltpu.VMEM((B_PER_W * WIN,), jnp.int32),
            pltpu.VMEM((ROWS_PER_W,), jnp.int32),
            pltpu.VMEM((ROWS_PER_W,), jnp.int32),
            pltpu.VMEM((2, CHUNK, EMB), jnp.float32),
            pltpu.SemaphoreType.DMA,
            pltpu.SemaphoreType.DMA,
            pltpu.SemaphoreType.DMA,
        ],
        compiler_params=pltpu.CompilerParams(
            use_tc_tiling_on_sc=False, needs_layout_passes=False
        ),
    )
    def _sc_gather(x_hbm, table_hbm, pat_hbm, out_hbm,
                   x_v, pat_v, idx_v, rows_v, gsem, osem0, osem1):
        wid = lax.axis_index("s") * 2 + lax.axis_index("c")
        pltpu.sync_copy(x_hbm.at[pl.ds(wid * B_PER_W * WIN, B_PER_W * WIN)], x_v)
        pltpu.sync_copy(pat_hbm, pat_v)
        for i in range(ROWS_PER_W // 16):
            sel = pat_v[pl.ds(16 * i, 16)]
            vals = plsc.load_gather(x_v, [sel])
            idx_v[pl.ds(16 * i, 16)] = vals
        osems = (osem0, osem1)
        base = wid * ROWS_PER_W
        for j in range(NCHUNK):
            slot = j % 2
            buf = rows_v.at[slot]
            gather = pltpu.async_copy(
                table_hbm.at[idx_v.at[pl.ds(j * CHUNK, CHUNK)]], buf, gsem
            )
            if j >= 2:
                pltpu.make_async_copy(
                    rows_v.at[slot],
                    out_hbm.at[pl.ds(base + (j - 2) * CHUNK, CHUNK)],
                    osems[slot],
                ).wait()
            gather.wait()
            pltpu.async_copy(
                buf, out_hbm.at[pl.ds(base + j * CHUNK, CHUNK)], osems[slot]
            )
        for j in (NCHUNK - 2, NCHUNK - 1):
            slot = j % 2
            pltpu.make_async_copy(
                rows_v.at[slot], out_hbm.at[pl.ds(base + j * CHUNK, CHUNK)],
                osems[slot],
            ).wait()

    return _sc_gather


def _mlp_body(flat_ref, w1_ref, b1_ref, w2_ref, b2_ref, out_ref):
    nrow = flat_ref.shape[0] * 8
    acc = b1_ref[...]
    for t in range(3):
        xt = flat_ref[:, t, :, :].reshape(nrow, 128)
        acc = acc + jnp.dot(xt, w1_ref[t], preferred_element_type=jnp.float32)
    h = jnp.tanh(acc)
    out_ref[...] = (
        jnp.dot(h, w2_ref[...], preferred_element_type=jnp.float32) + b2_ref[...]
    )


BLKR = 256


@jax.jit
def kernel(x, table, W1, b1, W2, b2):
    pat = jnp.asarray(_PATTERN)
    flat = _build_sc_gather()(x.reshape(B * WIN), table, pat)
    flat4d = flat.reshape(B // 8, 3, 8, 128)
    w1p = jnp.concatenate([W1, jnp.zeros((64, HID), jnp.float32)], axis=0)
    w1p = w1p.reshape(3, 128, HID)
    out = pl.pallas_call(
        _mlp_body,
        grid=(B // (8 * BLKR),),
        in_specs=[
            pl.BlockSpec((BLKR, 3, 8, 128), lambda i: (i, 0, 0, 0)),
            pl.BlockSpec((3, 128, HID), lambda i: (0, 0, 0)),
            pl.BlockSpec((1, HID), lambda i: (0, 0)),
            pl.BlockSpec((HID, OUT), lambda i: (0, 0)),
            pl.BlockSpec((1, OUT), lambda i: (0, 0)),
        ],
        out_specs=pl.BlockSpec((8 * BLKR, OUT), lambda i: (i, 0)),
        out_shape=jax.ShapeDtypeStruct((B, OUT), jnp.float32),
    )(flat4d, w1p, b1.reshape(1, HID), W2, b2.reshape(1, OUT))
    return out
